# baseline (speedup 1.0000x reference)
import functools

import jax
import jax.numpy as jnp
from jax import lax
from jax.experimental import pallas as pl
from jax.experimental.pallas import tpu as pltpu
from jax.experimental.pallas import tpu_sc as plsc

B = 4096
L = 200
D = 64
C0 = 128
C1 = L - C0
NW = 32
BPW = B // NW
NV = D // 16
NBUF = 4


def _make_sc_sums():
    mesh = plsc.VectorSubcoreMesh(core_axis_name="c", subcore_axis_name="s")

    @functools.partial(
        pl.kernel,
        out_type=jax.ShapeDtypeStruct((B * D,), jnp.float32),
        mesh=mesh,
        compiler_params=pltpu.CompilerParams(use_tc_tiling_on_sc=False),
        scratch_types=(
            [pltpu.VMEM((BPW * L,), jnp.int32)]
            + [pltpu.VMEM((L, D), jnp.float32) for _ in range(NBUF)]
            + [pltpu.VMEM((BPW * D,), jnp.float32)]
            + [pltpu.SemaphoreType.DMA for _ in range(NBUF)]
        ),
    )
    def sc_sums(emb_hbm, out_hbm, idx_v, *rest):
        bufs = rest[:NBUF]
        out_v = rest[NBUF]
        sems = rest[NBUF + 1 :]

        wid = lax.axis_index("s") * 2 + lax.axis_index("c")

        def zfill(i, c):
            idx_v[pl.ds(i * 16, 16)] = jnp.zeros((16,), jnp.int32)
            return c

        lax.fori_loop(0, BPW * L // 16, zfill, 0)

        def fire(s, b):
            pltpu.async_copy(
                emb_hbm.at[idx_v.at[pl.ds(b * L, C0)]],
                bufs[s].at[pl.ds(0, C0)],
                sems[s],
            )
            pltpu.async_copy(
                emb_hbm.at[idx_v.at[pl.ds(b * L + C0, C1)]],
                bufs[s].at[pl.ds(C0, C1)],
                sems[s],
            )

        def drain(s):
            pltpu.make_async_copy(
                emb_hbm.at[idx_v.at[pl.ds(0, C0)]],
                bufs[s].at[pl.ds(0, C0)],
                sems[s],
            ).wait()
            pltpu.make_async_copy(
                emb_hbm.at[idx_v.at[pl.ds(0, C1)]],
                bufs[s].at[pl.ds(C0, C1)],
                sems[s],
            ).wait()

        for s in range(NBUF):
            fire(s, s)

        def group(g, carry):
            for k in range(NBUF):
                b = g * NBUF + k
                drain(k)
                zero = jnp.zeros((16,), jnp.float32)
                buf = bufs[k]

                def tok(t, acc, buf=buf):
                    return tuple(
                        acc[j] + buf[t, pl.ds(16 * j, 16)] for j in range(NV)
                    ) + tuple(
                        acc[NV + j] + buf[L // 2 + t, pl.ds(16 * j, 16)]
                        for j in range(NV)
                    )

                acc = lax.fori_loop(0, L // 2, tok, (zero,) * (2 * NV), unroll=2)
                for j in range(NV):
                    out_v[pl.ds(b * D + 16 * j, 16)] = acc[j] + acc[NV + j]

                @pl.when(b + NBUF < BPW)
                def _(k=k, b=b):
                    fire(k, b + NBUF)

            return carry

        lax.fori_loop(0, BPW // NBUF, group, 0)
        pltpu.sync_copy(out_v, out_hbm.at[pl.ds(wid * (BPW * D), BPW * D)])

    return sc_sums


_sc_sums_cache = []


def _get_sc_sums():
    if not _sc_sums_cache:
        _sc_sums_cache.append(_make_sc_sums())
    return _sc_sums_cache[0]


def _tc_head_body(x_ref, s_ref, w1_ref, b1_ref, w2_ref, b2_ref, o_ref):
    cnt = jnp.sum((x_ref[...] != 0).astype(jnp.float32), axis=1, keepdims=True)
    pooled = s_ref[...] / jnp.maximum(cnt, 1.0)
    h = jnp.maximum(
        jnp.dot(pooled, w1_ref[...], preferred_element_type=jnp.float32)
        + b1_ref[...],
        0.0,
    )
    o_ref[...] = (
        jnp.dot(h, w2_ref[...], preferred_element_type=jnp.float32) + b2_ref[...]
    )


def _tc_head(x, sums, W1, b1r, W2p, b2r):
    blk = 1024
    return pl.pallas_call(
        _tc_head_body,
        out_shape=jax.ShapeDtypeStruct((B, 128), jnp.float32),
        grid=(B // blk,),
        in_specs=[
            pl.BlockSpec((blk, L), lambda i: (i, 0)),
            pl.BlockSpec((blk, D), lambda i: (i, 0)),
            pl.BlockSpec((D, 128), lambda i: (0, 0)),
            pl.BlockSpec((1, 128), lambda i: (0, 0)),
            pl.BlockSpec((128, 128), lambda i: (0, 0)),
            pl.BlockSpec((1, 128), lambda i: (0, 0)),
        ],
        out_specs=pl.BlockSpec((blk, 128), lambda i: (i, 0)),
    )(x, sums, W1, b1r, W2p, b2r)


def kernel(x, emb, W1, b1, W2, b2):
    x = x.astype(jnp.int32)
    nc = W2.shape[1]
    sums1d = _get_sc_sums()(emb)
    return sums1d[: B * nc].reshape(B, nc)


# speedup vs baseline: 23.2197x; 23.2197x over previous
import functools

import jax
import jax.numpy as jnp
from jax import lax
from jax.experimental import pallas as pl
from jax.experimental.pallas import tpu as pltpu
from jax.experimental.pallas import tpu_sc as plsc

B = 4096
L = 200
D = 64
C0 = 128
C1 = L - C0
NW = 32
BPW = B // NW
NV = D // 16
NBUF = 4
VOCAB_HALF = 500000


def _make_sc_sums():
    mesh = plsc.VectorSubcoreMesh(core_axis_name="c", subcore_axis_name="s")

    @functools.partial(
        pl.kernel,
        out_type=jax.ShapeDtypeStruct((B * D,), jnp.float32),
        mesh=mesh,
        compiler_params=pltpu.CompilerParams(use_tc_tiling_on_sc=False),
        scratch_types=(
            [pltpu.VMEM((BPW * L,), jnp.int32)]
            + [pltpu.VMEM((L, D), jnp.float32) for _ in range(NBUF)]
            + [pltpu.VMEM((BPW * D,), jnp.float32)]
            + [pltpu.SemaphoreType.DMA for _ in range(NBUF)]
        ),
    )
    def sc_sums(x_hbm, emb_hbm, out_hbm, idx_v, *rest):
        bufs = rest[:NBUF]
        out_v = rest[NBUF]
        sems = rest[NBUF + 1 :]

        wid = lax.axis_index("s") * 2 + lax.axis_index("c")
        pltpu.sync_copy(x_hbm.at[pl.ds(wid * (BPW * L), BPW * L)], idx_v)

        def fire(s, b):
            pltpu.async_copy(
                emb_hbm.at[idx_v.at[pl.ds(b * L, C0)]],
                bufs[s].at[pl.ds(0, C0)],
                sems[s],
            )
            pltpu.async_copy(
                emb_hbm.at[idx_v.at[pl.ds(b * L + C0, C1)]],
                bufs[s].at[pl.ds(C0, C1)],
                sems[s],
            )

        def drain(s):
            pltpu.make_async_copy(
                emb_hbm.at[idx_v.at[pl.ds(0, C0)]],
                bufs[s].at[pl.ds(0, C0)],
                sems[s],
            ).wait()
            pltpu.make_async_copy(
                emb_hbm.at[idx_v.at[pl.ds(0, C1)]],
                bufs[s].at[pl.ds(C0, C1)],
                sems[s],
            ).wait()

        for s in range(NBUF):
            fire(s, s)

        def group(g, carry):
            for k in range(NBUF):
                b = g * NBUF + k
                drain(k)
                zero = jnp.zeros((16,), jnp.float32)
                buf = bufs[k]

                def tok(t, acc, buf=buf):
                    return tuple(
                        acc[j] + buf[t, pl.ds(16 * j, 16)] for j in range(NV)
                    ) + tuple(
                        acc[NV + j] + buf[L // 2 + t, pl.ds(16 * j, 16)]
                        for j in range(NV)
                    )

                acc = lax.fori_loop(0, L // 2, tok, (zero,) * (2 * NV), unroll=2)
                for j in range(NV):
                    out_v[pl.ds(b * D + 16 * j, 16)] = acc[j] + acc[NV + j]

                @pl.when(b + NBUF < BPW)
                def _(k=k, b=b):
                    fire(k, b + NBUF)

            return carry

        lax.fori_loop(0, BPW // NBUF, group, 0)
        pltpu.sync_copy(out_v, out_hbm.at[pl.ds(wid * (BPW * D), BPW * D)])

    return sc_sums


_sc_sums_cache = []


def _get_sc_sums():
    if not _sc_sums_cache:
        _sc_sums_cache.append(_make_sc_sums())
    return _sc_sums_cache[0]


def _tc_head_body(x_ref, s_ref, w1_ref, b1_ref, w2_ref, b2_ref, o_ref):
    cnt = jnp.sum((x_ref[...] != 0).astype(jnp.float32), axis=1, keepdims=True)
    pooled = s_ref[...] / jnp.maximum(cnt, 1.0)
    h = jnp.maximum(
        jnp.dot(pooled, w1_ref[...], preferred_element_type=jnp.float32)
        + b1_ref[...],
        0.0,
    )
    o_ref[...] = (
        jnp.dot(h, w2_ref[...], preferred_element_type=jnp.float32) + b2_ref[...]
    )


def _tc_head(x, sums, W1, b1r, W2p, b2r):
    blk = 1024
    return pl.pallas_call(
        _tc_head_body,
        out_shape=jax.ShapeDtypeStruct((B, 128), jnp.float32),
        grid=(B // blk,),
        in_specs=[
            pl.BlockSpec((blk, L), lambda i: (i, 0)),
            pl.BlockSpec((blk, D), lambda i: (i, 0)),
            pl.BlockSpec((D, 128), lambda i: (0, 0)),
            pl.BlockSpec((1, 128), lambda i: (0, 0)),
            pl.BlockSpec((128, 128), lambda i: (0, 0)),
            pl.BlockSpec((1, 128), lambda i: (0, 0)),
        ],
        out_specs=pl.BlockSpec((blk, 128), lambda i: (i, 0)),
    )(x, sums, W1, b1r, W2p, b2r)


def kernel(x, emb, W1, b1, W2, b2):
    x = x.astype(jnp.int32)
    nc = W2.shape[1]
    x_lin = jax.lax.optimization_barrier(x.reshape(B * L // 128, 128))
    emb_lin = jax.lax.optimization_barrier(emb.reshape(VOCAB_HALF, 128)).reshape(
        emb.shape
    )
    sums = _get_sc_sums()(x_lin.reshape(-1), emb_lin).reshape(B, D)
    W2p = jnp.pad(W2, ((0, 0), (0, 128 - nc)))
    b2r = jnp.pad(b2, ((0, 128 - nc),)).reshape(1, 128)
    b1r = b1.reshape(1, 128)
    out = _tc_head(x, sums, W1, b1r, W2p, b2r)
    return out[:, :nc]


# speedup vs baseline: 25.4565x; 1.0963x over previous
import functools

import jax
import jax.numpy as jnp
from jax import lax
from jax.experimental import pallas as pl
from jax.experimental.pallas import tpu as pltpu
from jax.experimental.pallas import tpu_sc as plsc

B = 4096
L = 200
D = 64
C0 = 128
C1 = L - C0
NW = 32
BPW = B // NW
NV = D // 16
NBUF = 3
T1 = 499712
T2 = 2 * T1
VP = 500736


def _tc_pack_body(a_ref, b_ref, o_ref):
    o_ref[:, 0:64] = jnp.swapaxes(a_ref[...], 0, 1)
    o_ref[:, 64:128] = jnp.swapaxes(b_ref[...], 0, 1)


def _tc_pack(emb64):
    cb = 1024
    return pl.pallas_call(
        _tc_pack_body,
        out_shape=jax.ShapeDtypeStruct((VP, 128), jnp.float32),
        grid=(VP // cb,),
        in_specs=[
            pl.BlockSpec((D, cb), lambda i: (0, jnp.where(i < 488, i, 976))),
            pl.BlockSpec((D, cb), lambda i: (0, i + 488)),
        ],
        out_specs=pl.BlockSpec((cb, 128), lambda i: (i, 0)),
    )(emb64, emb64)


def _make_sc_sums():
    mesh = plsc.VectorSubcoreMesh(core_axis_name="c", subcore_axis_name="s")

    @functools.partial(
        pl.kernel,
        out_type=jax.ShapeDtypeStruct((B * D,), jnp.float32),
        mesh=mesh,
        compiler_params=pltpu.CompilerParams(use_tc_tiling_on_sc=False),
        scratch_types=(
            [pltpu.VMEM((BPW * L + 16,), jnp.int32)]
            + [pltpu.VMEM((L, 128), jnp.float32) for _ in range(NBUF)]
            + [pltpu.VMEM((C0,), jnp.int32) for _ in range(NBUF)]
            + [pltpu.VMEM((C1,), jnp.int32) for _ in range(NBUF)]
            + [pltpu.VMEM((BPW * D,), jnp.float32)]
            + [pltpu.SemaphoreType.DMA for _ in range(NBUF)]
        ),
    )
    def sc_sums(x_hbm, emb_hbm, out_hbm, idx_v, *rest):
        bufs = rest[:NBUF]
        idxa = rest[NBUF : 2 * NBUF]
        idxb = rest[2 * NBUF : 3 * NBUF]
        out_v = rest[3 * NBUF]
        sems = rest[3 * NBUF + 1 :]

        wid = lax.axis_index("s") * 2 + lax.axis_index("c")
        pltpu.sync_copy(
            x_hbm.at[pl.ds(wid * (BPW * L), BPW * L)], idx_v.at[pl.ds(0, BPW * L)]
        )

        def fire(s, b):
            ia, ib = idxa[s], idxb[s]
            for c in range(C0 // 16):
                v = idx_v[pl.ds(b * L + 16 * c, 16)]
                ia[pl.ds(16 * c, 16)] = v - jnp.where(v >= T1, T1, 0)
            for off in (0, 16, 32, 48, C1 - 16):
                v = idx_v[pl.ds(b * L + C0 + off, 16)]
                ib[pl.ds(off, 16)] = v - jnp.where(v >= T1, T1, 0)
            pltpu.async_copy(
                emb_hbm.at[ia], bufs[s].at[pl.ds(0, C0)], sems[s]
            )
            pltpu.async_copy(
                emb_hbm.at[ib], bufs[s].at[pl.ds(C0, C1)], sems[s]
            )

        def drain(s):
            pltpu.make_async_copy(
                emb_hbm.at[idxa[s]], bufs[s].at[pl.ds(0, C0)], sems[s]
            ).wait()
            pltpu.make_async_copy(
                emb_hbm.at[idxb[s]], bufs[s].at[pl.ds(C0, C1)], sems[s]
            ).wait()

        def accum_row(s, b):
            drain(s)
            zero = jnp.zeros((16,), jnp.float32)
            buf = bufs[s]
            bt = b * L

            def tok(t, acc, buf=buf, bt=bt):
                s0 = idx_v[pl.ds(bt + t, 16)][0]
                s1 = idx_v[pl.ds(bt + L // 2 + t, 16)][0]
                o0 = jnp.where((s0 >= T1) & (s0 < T2), 64, 0)
                o1 = jnp.where((s1 >= T1) & (s1 < T2), 64, 0)
                return tuple(
                    acc[j] + buf[t, pl.ds(o0 + 16 * j, 16)] for j in range(NV)
                ) + tuple(
                    acc[NV + j] + buf[L // 2 + t, pl.ds(o1 + 16 * j, 16)]
                    for j in range(NV)
                )

            acc = lax.fori_loop(0, L // 2, tok, (zero,) * (2 * NV), unroll=2)
            for j in range(NV):
                out_v[pl.ds(b * D + 16 * j, 16)] = acc[j] + acc[NV + j]

        for s in range(NBUF):
            fire(s, s)

        def group(g, carry):
            for k in range(NBUF):
                b = g * NBUF + k
                accum_row(k, b)

                @pl.when(b + NBUF < BPW)
                def _(k=k, b=b):
                    fire(k, b + NBUF)

            return carry

        ng = BPW // NBUF
        lax.fori_loop(0, ng, group, 0)
        for i, b in enumerate(range(ng * NBUF, BPW)):
            accum_row(b % NBUF, b)

        pltpu.sync_copy(out_v, out_hbm.at[pl.ds(wid * (BPW * D), BPW * D)])

    return sc_sums


_sc_sums_cache = []


def _get_sc_sums():
    if not _sc_sums_cache:
        _sc_sums_cache.append(_make_sc_sums())
    return _sc_sums_cache[0]


def _tc_head_body(x_ref, s_ref, w1_ref, b1_ref, w2_ref, b2_ref, o_ref):
    cnt = jnp.sum((x_ref[...] != 0).astype(jnp.float32), axis=1, keepdims=True)
    pooled = s_ref[...] / jnp.maximum(cnt, 1.0)
    h = jnp.maximum(
        jnp.dot(pooled, w1_ref[...], preferred_element_type=jnp.float32)
        + b1_ref[...],
        0.0,
    )
    o_ref[...] = (
        jnp.dot(h, w2_ref[...], preferred_element_type=jnp.float32) + b2_ref[...]
    )


def _tc_head(x, sums, W1, b1r, W2p, b2r):
    blk = 1024
    return pl.pallas_call(
        _tc_head_body,
        out_shape=jax.ShapeDtypeStruct((B, 128), jnp.float32),
        grid=(B // blk,),
        in_specs=[
            pl.BlockSpec((blk, L), lambda i: (i, 0)),
            pl.BlockSpec((blk, D), lambda i: (i, 0)),
            pl.BlockSpec((D, 128), lambda i: (0, 0)),
            pl.BlockSpec((1, 128), lambda i: (0, 0)),
            pl.BlockSpec((128, 128), lambda i: (0, 0)),
            pl.BlockSpec((1, 128), lambda i: (0, 0)),
        ],
        out_specs=pl.BlockSpec((blk, 128), lambda i: (i, 0)),
    )(x, sums, W1, b1r, W2p, b2r)


def kernel(x, emb, W1, b1, W2, b2):
    x = x.astype(jnp.int32)
    nc = W2.shape[1]
    x_lin = jax.lax.optimization_barrier(x.reshape(B * L // 128, 128))
    emb_p = _tc_pack(jnp.swapaxes(emb, 0, 1))
    sums = _get_sc_sums()(x_lin.reshape(-1), emb_p).reshape(B, D)
    W2p = jnp.pad(W2, ((0, 0), (0, 128 - nc)))
    b2r = jnp.pad(b2, ((0, 128 - nc),)).reshape(1, 128)
    b1r = b1.reshape(1, 128)
    out = _tc_head(x, sums, W1, b1r, W2p, b2r)
    return out[:, :nc]


# speedup vs baseline: 35.8740x; 1.4092x over previous
import functools

import jax
import jax.numpy as jnp
from jax import lax
from jax.experimental import pallas as pl
from jax.experimental.pallas import tpu as pltpu
from jax.experimental.pallas import tpu_sc as plsc

B = 4096
L = 200
D = 64
C0 = 128
C1 = L - C0
NW = 32
BPW = B // NW
NV = D // 16
NBUF = 3
T1 = 499712
T2 = 2 * T1
PCB = 4096
NHB = T1 // PCB
TBL = 1000000 // PCB
VP = PCB * (NHB + 1)


def _tc_pack_body(a_ref, b_ref, o_ref):
    o_ref[:, 0:64] = jnp.swapaxes(a_ref[...], 0, 1)
    o_ref[:, 64:128] = jnp.swapaxes(b_ref[...], 0, 1)


def _tc_pack(emb64):
    return pl.pallas_call(
        _tc_pack_body,
        out_shape=jax.ShapeDtypeStruct((VP, 128), jnp.float32),
        grid=(VP // PCB,),
        in_specs=[
            pl.BlockSpec((D, PCB), lambda i: (0, jnp.where(i < NHB, i, TBL))),
            pl.BlockSpec((D, PCB), lambda i: (0, i + NHB)),
        ],
        out_specs=pl.BlockSpec((PCB, 128), lambda i: (i, 0)),
    )(emb64, emb64)


def _make_sc_sums():
    mesh = plsc.VectorSubcoreMesh(core_axis_name="c", subcore_axis_name="s")

    @functools.partial(
        pl.kernel,
        out_type=jax.ShapeDtypeStruct((B * D,), jnp.float32),
        mesh=mesh,
        compiler_params=pltpu.CompilerParams(use_tc_tiling_on_sc=False),
        scratch_types=(
            [pltpu.VMEM((BPW * L + 16,), jnp.int32)]
            + [pltpu.VMEM((L, 128), jnp.float32) for _ in range(NBUF)]
            + [pltpu.VMEM((C0,), jnp.int32) for _ in range(NBUF)]
            + [pltpu.VMEM((C1,), jnp.int32) for _ in range(NBUF)]
            + [pltpu.VMEM((BPW * D,), jnp.float32)]
            + [pltpu.SemaphoreType.DMA for _ in range(NBUF)]
        ),
    )
    def sc_sums(x_hbm, emb_hbm, out_hbm, idx_v, *rest):
        bufs = rest[:NBUF]
        idxa = rest[NBUF : 2 * NBUF]
        idxb = rest[2 * NBUF : 3 * NBUF]
        out_v = rest[3 * NBUF]
        sems = rest[3 * NBUF + 1 :]

        wid = lax.axis_index("s") * 2 + lax.axis_index("c")
        pltpu.sync_copy(
            x_hbm.at[pl.ds(wid * (BPW * L), BPW * L)], idx_v.at[pl.ds(0, BPW * L)]
        )

        def fire(s, b):
            ia, ib = idxa[s], idxb[s]
            for c in range(C0 // 16):
                v = idx_v[pl.ds(b * L + 16 * c, 16)]
                ia[pl.ds(16 * c, 16)] = v - jnp.where(v >= T1, T1, 0)
            for off in (0, 16, 32, 48, C1 - 16):
                v = idx_v[pl.ds(b * L + C0 + off, 16)]
                ib[pl.ds(off, 16)] = v - jnp.where(v >= T1, T1, 0)
            pltpu.async_copy(
                emb_hbm.at[ia], bufs[s].at[pl.ds(0, C0)], sems[s]
            )
            pltpu.async_copy(
                emb_hbm.at[ib], bufs[s].at[pl.ds(C0, C1)], sems[s]
            )

        def drain(s):
            pltpu.make_async_copy(
                emb_hbm.at[idxa[s]], bufs[s].at[pl.ds(0, C0)], sems[s]
            ).wait()
            pltpu.make_async_copy(
                emb_hbm.at[idxb[s]], bufs[s].at[pl.ds(C0, C1)], sems[s]
            ).wait()

        def accum_row(s, b):
            drain(s)
            zero = jnp.zeros((16,), jnp.float32)
            buf = bufs[s]
            bt = b * L

            def tok(t, acc, buf=buf, bt=bt):
                s0 = idx_v[pl.ds(bt + t, 16)][0]
                s1 = idx_v[pl.ds(bt + L // 2 + t, 16)][0]
                o0 = jnp.where((s0 >= T1) & (s0 < T2), 64, 0)
                o1 = jnp.where((s1 >= T1) & (s1 < T2), 64, 0)
                return tuple(
                    acc[j] + buf[t, pl.ds(o0 + 16 * j, 16)] for j in range(NV)
                ) + tuple(
                    acc[NV + j] + buf[L // 2 + t, pl.ds(o1 + 16 * j, 16)]
                    for j in range(NV)
                )

            acc = lax.fori_loop(0, L // 2, tok, (zero,) * (2 * NV), unroll=2)
            for j in range(NV):
                out_v[pl.ds(b * D + 16 * j, 16)] = acc[j] + acc[NV + j]

        for s in range(NBUF):
            fire(s, s)

        def group(g, carry):
            for k in range(NBUF):
                b = g * NBUF + k
                accum_row(k, b)

                @pl.when(b + NBUF < BPW)
                def _(k=k, b=b):
                    fire(k, b + NBUF)

            return carry

        ng = BPW // NBUF
        lax.fori_loop(0, ng, group, 0)
        for i, b in enumerate(range(ng * NBUF, BPW)):
            accum_row(b % NBUF, b)

        pltpu.sync_copy(out_v, out_hbm.at[pl.ds(wid * (BPW * D), BPW * D)])

    return sc_sums


_sc_sums_cache = []


def _get_sc_sums():
    if not _sc_sums_cache:
        _sc_sums_cache.append(_make_sc_sums())
    return _sc_sums_cache[0]


def _tc_head_body(x_ref, s_ref, w1_ref, b1_ref, w2_ref, b2_ref, o_ref):
    cnt = jnp.sum((x_ref[...] != 0).astype(jnp.float32), axis=1, keepdims=True)
    pooled = s_ref[...] / jnp.maximum(cnt, 1.0)
    h = jnp.maximum(
        jnp.dot(pooled, w1_ref[...], preferred_element_type=jnp.float32)
        + b1_ref[...],
        0.0,
    )
    o_ref[...] = (
        jnp.dot(h, w2_ref[...], preferred_element_type=jnp.float32) + b2_ref[...]
    )


def _tc_head(x, sums, W1, b1r, W2p, b2r):
    blk = 1024
    return pl.pallas_call(
        _tc_head_body,
        out_shape=jax.ShapeDtypeStruct((B, 128), jnp.float32),
        grid=(B // blk,),
        in_specs=[
            pl.BlockSpec((blk, L), lambda i: (i, 0)),
            pl.BlockSpec((blk, D), lambda i: (i, 0)),
            pl.BlockSpec((D, 128), lambda i: (0, 0)),
            pl.BlockSpec((1, 128), lambda i: (0, 0)),
            pl.BlockSpec((128, 128), lambda i: (0, 0)),
            pl.BlockSpec((1, 128), lambda i: (0, 0)),
        ],
        out_specs=pl.BlockSpec((blk, 128), lambda i: (i, 0)),
    )(x, sums, W1, b1r, W2p, b2r)


def kernel(x, emb, W1, b1, W2, b2):
    x = x.astype(jnp.int32)
    nc = W2.shape[1]
    x_lin = jax.lax.optimization_barrier(x.reshape(B * L // 128, 128))
    emb_p = _tc_pack(jnp.swapaxes(emb, 0, 1))
    sums = _get_sc_sums()(x_lin.reshape(-1), emb_p).reshape(B, D)
    W2p = jnp.pad(W2, ((0, 0), (0, 128 - nc)))
    b2r = jnp.pad(b2, ((0, 128 - nc),)).reshape(1, 128)
    b1r = b1.reshape(1, 128)
    out = _tc_head(x, sums, W1, b1r, W2p, b2r)
    return out[:, :nc]


# speedup vs baseline: 38.7477x; 1.0801x over previous
import functools

import jax
import jax.numpy as jnp
from jax import lax
from jax.experimental import pallas as pl
from jax.experimental.pallas import tpu as pltpu
from jax.experimental.pallas import tpu_sc as plsc

B = 4096
L = 200
D = 64
C0 = 128
C1 = L - C0
NW = 32
BPW = B // NW
NV = D // 16
NBUF = 3
T1 = 499712
T2 = 2 * T1
PCB = 8192
NHB = T1 // PCB
TBL = 1000000 // PCB
VP = PCB * (NHB + 1)


def _tc_pack_body(a_ref, b_ref, o_ref):
    o_ref[:, 0:64] = jnp.swapaxes(a_ref[...], 0, 1)
    o_ref[:, 64:128] = jnp.swapaxes(b_ref[...], 0, 1)


def _tc_pack(emb64):
    return pl.pallas_call(
        _tc_pack_body,
        out_shape=jax.ShapeDtypeStruct((VP, 128), jnp.float32),
        grid=(VP // PCB,),
        in_specs=[
            pl.BlockSpec((D, PCB), lambda i: (0, jnp.where(i < NHB, i, TBL))),
            pl.BlockSpec((D, PCB), lambda i: (0, i + NHB)),
        ],
        out_specs=pl.BlockSpec((PCB, 128), lambda i: (i, 0)),
    )(emb64, emb64)


def _make_sc_sums():
    mesh = plsc.VectorSubcoreMesh(core_axis_name="c", subcore_axis_name="s")

    @functools.partial(
        pl.kernel,
        out_type=jax.ShapeDtypeStruct((B * D,), jnp.float32),
        mesh=mesh,
        compiler_params=pltpu.CompilerParams(use_tc_tiling_on_sc=False),
        scratch_types=(
            [pltpu.VMEM((BPW * L + 16,), jnp.int32)]
            + [pltpu.VMEM((L, 128), jnp.float32) for _ in range(NBUF)]
            + [pltpu.VMEM((C0,), jnp.int32) for _ in range(NBUF)]
            + [pltpu.VMEM((C1,), jnp.int32) for _ in range(NBUF)]
            + [pltpu.VMEM((BPW * D,), jnp.float32)]
            + [pltpu.SemaphoreType.DMA for _ in range(NBUF)]
        ),
    )
    def sc_sums(x_hbm, emb_hbm, out_hbm, idx_v, *rest):
        bufs = rest[:NBUF]
        idxa = rest[NBUF : 2 * NBUF]
        idxb = rest[2 * NBUF : 3 * NBUF]
        out_v = rest[3 * NBUF]
        sems = rest[3 * NBUF + 1 :]

        wid = lax.axis_index("s") * 2 + lax.axis_index("c")
        pltpu.sync_copy(
            x_hbm.at[pl.ds(wid * (BPW * L), BPW * L)], idx_v.at[pl.ds(0, BPW * L)]
        )

        def fire(s, b):
            ia, ib = idxa[s], idxb[s]
            for c in range(C0 // 16):
                v = idx_v[pl.ds(b * L + 16 * c, 16)]
                ia[pl.ds(16 * c, 16)] = v - jnp.where(v >= T1, T1, 0)
            for off in (0, 16, 32, 48, C1 - 16):
                v = idx_v[pl.ds(b * L + C0 + off, 16)]
                ib[pl.ds(off, 16)] = v - jnp.where(v >= T1, T1, 0)
            pltpu.async_copy(
                emb_hbm.at[ia], bufs[s].at[pl.ds(0, C0)], sems[s]
            )
            pltpu.async_copy(
                emb_hbm.at[ib], bufs[s].at[pl.ds(C0, C1)], sems[s]
            )

        def drain(s):
            pltpu.make_async_copy(
                emb_hbm.at[idxa[s]], bufs[s].at[pl.ds(0, C0)], sems[s]
            ).wait()
            pltpu.make_async_copy(
                emb_hbm.at[idxb[s]], bufs[s].at[pl.ds(C0, C1)], sems[s]
            ).wait()

        def accum_row(s, b):
            drain(s)
            zero = jnp.zeros((16,), jnp.float32)
            buf = bufs[s]
            bt = b * L

            def tok(t, acc, buf=buf, bt=bt):
                s0 = idx_v[pl.ds(bt + t, 16)][0]
                s1 = idx_v[pl.ds(bt + L // 2 + t, 16)][0]
                o0 = jnp.where((s0 >= T1) & (s0 < T2), 64, 0)
                o1 = jnp.where((s1 >= T1) & (s1 < T2), 64, 0)
                return tuple(
                    acc[j] + buf[t, pl.ds(o0 + 16 * j, 16)] for j in range(NV)
                ) + tuple(
                    acc[NV + j] + buf[L // 2 + t, pl.ds(o1 + 16 * j, 16)]
                    for j in range(NV)
                )

            acc = lax.fori_loop(0, L // 2, tok, (zero,) * (2 * NV), unroll=2)
            for j in range(NV):
                out_v[pl.ds(b * D + 16 * j, 16)] = acc[j] + acc[NV + j]

        for s in range(NBUF):
            fire(s, s)

        def group(g, carry):
            for k in range(NBUF):
                b = g * NBUF + k
                accum_row(k, b)

                @pl.when(b + NBUF < BPW)
                def _(k=k, b=b):
                    fire(k, b + NBUF)

            return carry

        ng = BPW // NBUF
        lax.fori_loop(0, ng, group, 0)
        for i, b in enumerate(range(ng * NBUF, BPW)):
            accum_row(b % NBUF, b)

        pltpu.sync_copy(out_v, out_hbm.at[pl.ds(wid * (BPW * D), BPW * D)])

    return sc_sums


_sc_sums_cache = []


def _get_sc_sums():
    if not _sc_sums_cache:
        _sc_sums_cache.append(_make_sc_sums())
    return _sc_sums_cache[0]


def _tc_head_body(x_ref, s_ref, w1_ref, b1_ref, w2_ref, b2_ref, o_ref):
    cnt = jnp.sum((x_ref[...] != 0).astype(jnp.float32), axis=1, keepdims=True)
    pooled = s_ref[...] / jnp.maximum(cnt, 1.0)
    h = jnp.maximum(
        jnp.dot(pooled, w1_ref[...], preferred_element_type=jnp.float32)
        + b1_ref[...],
        0.0,
    )
    o_ref[...] = (
        jnp.dot(h, w2_ref[...], preferred_element_type=jnp.float32) + b2_ref[...]
    )


def _tc_head(x, sums, W1, b1r, W2p, b2r):
    blk = 1024
    return pl.pallas_call(
        _tc_head_body,
        out_shape=jax.ShapeDtypeStruct((B, 128), jnp.float32),
        grid=(B // blk,),
        in_specs=[
            pl.BlockSpec((blk, L), lambda i: (i, 0)),
            pl.BlockSpec((blk, D), lambda i: (i, 0)),
            pl.BlockSpec((D, 128), lambda i: (0, 0)),
            pl.BlockSpec((1, 128), lambda i: (0, 0)),
            pl.BlockSpec((128, 128), lambda i: (0, 0)),
            pl.BlockSpec((1, 128), lambda i: (0, 0)),
        ],
        out_specs=pl.BlockSpec((blk, 128), lambda i: (i, 0)),
    )(x, sums, W1, b1r, W2p, b2r)


def kernel(x, emb, W1, b1, W2, b2):
    x = x.astype(jnp.int32)
    nc = W2.shape[1]
    x_lin = jax.lax.optimization_barrier(x.reshape(B * L // 128, 128))
    emb_p = _tc_pack(jnp.swapaxes(emb, 0, 1))
    sums = _get_sc_sums()(x_lin.reshape(-1), emb_p).reshape(B, D)
    W2p = jnp.pad(W2, ((0, 0), (0, 128 - nc)))
    b2r = jnp.pad(b2, ((0, 128 - nc),)).reshape(1, 128)
    b1r = b1.reshape(1, 128)
    out = _tc_head(x, sums, W1, b1r, W2p, b2r)
    return out[:, :nc]


# speedup vs baseline: 39.4415x; 1.0179x over previous
import functools

import jax
import jax.numpy as jnp
from jax import lax
from jax.experimental import pallas as pl
from jax.experimental.pallas import tpu as pltpu
from jax.experimental.pallas import tpu_sc as plsc

B = 4096
L = 200
D = 64
C0 = 128
C1 = L - C0
NW = 32
BPW = B // NW
NV = D // 16
NBUF = 3
T1 = 491520
T2 = 2 * T1
PCB = 16384
NHB = T1 // PCB
TBL = 1000000 // PCB
VP = PCB * (NHB + 2)


def _tc_pack_body(a_ref, b_ref, o_ref):
    o_ref[:, 0:64] = jnp.swapaxes(a_ref[...], 0, 1)
    o_ref[:, 64:128] = jnp.swapaxes(b_ref[...], 0, 1)


def _tc_pack(emb64):
    return pl.pallas_call(
        _tc_pack_body,
        out_shape=jax.ShapeDtypeStruct((VP, 128), jnp.float32),
        grid=(VP // PCB,),
        in_specs=[
            pl.BlockSpec(
                (D, PCB),
                lambda i: (0, jnp.where(i < NHB, i, 2 * NHB + (i - NHB))),
            ),
            pl.BlockSpec((D, PCB), lambda i: (0, i + NHB)),
        ],
        out_specs=pl.BlockSpec((PCB, 128), lambda i: (i, 0)),
    )(emb64, emb64)


def _make_sc_sums():
    mesh = plsc.VectorSubcoreMesh(core_axis_name="c", subcore_axis_name="s")

    @functools.partial(
        pl.kernel,
        out_type=jax.ShapeDtypeStruct((B * D,), jnp.float32),
        mesh=mesh,
        compiler_params=pltpu.CompilerParams(use_tc_tiling_on_sc=False),
        scratch_types=(
            [pltpu.VMEM((BPW * L + 16,), jnp.int32)]
            + [pltpu.VMEM((L, 128), jnp.float32) for _ in range(NBUF)]
            + [pltpu.VMEM((C0,), jnp.int32) for _ in range(NBUF)]
            + [pltpu.VMEM((C1,), jnp.int32) for _ in range(NBUF)]
            + [pltpu.VMEM((BPW * D,), jnp.float32)]
            + [pltpu.SemaphoreType.DMA for _ in range(NBUF)]
        ),
    )
    def sc_sums(x_hbm, emb_hbm, out_hbm, idx_v, *rest):
        bufs = rest[:NBUF]
        idxa = rest[NBUF : 2 * NBUF]
        idxb = rest[2 * NBUF : 3 * NBUF]
        out_v = rest[3 * NBUF]
        sems = rest[3 * NBUF + 1 :]

        wid = lax.axis_index("s") * 2 + lax.axis_index("c")
        pltpu.sync_copy(
            x_hbm.at[pl.ds(wid * (BPW * L), BPW * L)], idx_v.at[pl.ds(0, BPW * L)]
        )

        def fire(s, b):
            ia, ib = idxa[s], idxb[s]
            for c in range(C0 // 16):
                v = idx_v[pl.ds(b * L + 16 * c, 16)]
                ia[pl.ds(16 * c, 16)] = v - jnp.where(v >= T1, T1, 0)
            for off in (0, 16, 32, 48, C1 - 16):
                v = idx_v[pl.ds(b * L + C0 + off, 16)]
                ib[pl.ds(off, 16)] = v - jnp.where(v >= T1, T1, 0)
            pltpu.async_copy(
                emb_hbm.at[ia], bufs[s].at[pl.ds(0, C0)], sems[s]
            )
            pltpu.async_copy(
                emb_hbm.at[ib], bufs[s].at[pl.ds(C0, C1)], sems[s]
            )

        def drain(s):
            pltpu.make_async_copy(
                emb_hbm.at[idxa[s]], bufs[s].at[pl.ds(0, C0)], sems[s]
            ).wait()
            pltpu.make_async_copy(
                emb_hbm.at[idxb[s]], bufs[s].at[pl.ds(C0, C1)], sems[s]
            ).wait()

        def accum_row(s, b):
            drain(s)
            zero = jnp.zeros((16,), jnp.float32)
            buf = bufs[s]
            bt = b * L

            def tok(t, acc, buf=buf, bt=bt):
                s0 = idx_v[pl.ds(bt + t, 16)][0]
                s1 = idx_v[pl.ds(bt + L // 2 + t, 16)][0]
                o0 = jnp.where((s0 >= T1) & (s0 < T2), 64, 0)
                o1 = jnp.where((s1 >= T1) & (s1 < T2), 64, 0)
                return tuple(
                    acc[j] + buf[t, pl.ds(o0 + 16 * j, 16)] for j in range(NV)
                ) + tuple(
                    acc[NV + j] + buf[L // 2 + t, pl.ds(o1 + 16 * j, 16)]
                    for j in range(NV)
                )

            acc = lax.fori_loop(0, L // 2, tok, (zero,) * (2 * NV), unroll=2)
            for j in range(NV):
                out_v[pl.ds(b * D + 16 * j, 16)] = acc[j] + acc[NV + j]

        for s in range(NBUF):
            fire(s, s)

        def group(g, carry):
            for k in range(NBUF):
                b = g * NBUF + k
                accum_row(k, b)

                @pl.when(b + NBUF < BPW)
                def _(k=k, b=b):
                    fire(k, b + NBUF)

            return carry

        ng = BPW // NBUF
        lax.fori_loop(0, ng, group, 0)
        for i, b in enumerate(range(ng * NBUF, BPW)):
            accum_row(b % NBUF, b)

        pltpu.sync_copy(out_v, out_hbm.at[pl.ds(wid * (BPW * D), BPW * D)])

    return sc_sums


_sc_sums_cache = []


def _get_sc_sums():
    if not _sc_sums_cache:
        _sc_sums_cache.append(_make_sc_sums())
    return _sc_sums_cache[0]


def _tc_head_body(x_ref, s_ref, w1_ref, b1_ref, w2_ref, b2_ref, o_ref):
    cnt = jnp.sum((x_ref[...] != 0).astype(jnp.float32), axis=1, keepdims=True)
    pooled = s_ref[...] / jnp.maximum(cnt, 1.0)
    h = jnp.maximum(
        jnp.dot(pooled, w1_ref[...], preferred_element_type=jnp.float32)
        + b1_ref[...],
        0.0,
    )
    o_ref[...] = (
        jnp.dot(h, w2_ref[...], preferred_element_type=jnp.float32) + b2_ref[...]
    )


def _tc_head(x, sums, W1, b1r, W2p, b2r):
    blk = 1024
    return pl.pallas_call(
        _tc_head_body,
        out_shape=jax.ShapeDtypeStruct((B, 128), jnp.float32),
        grid=(B // blk,),
        in_specs=[
            pl.BlockSpec((blk, L), lambda i: (i, 0)),
            pl.BlockSpec((blk, D), lambda i: (i, 0)),
            pl.BlockSpec((D, 128), lambda i: (0, 0)),
            pl.BlockSpec((1, 128), lambda i: (0, 0)),
            pl.BlockSpec((128, 128), lambda i: (0, 0)),
            pl.BlockSpec((1, 128), lambda i: (0, 0)),
        ],
        out_specs=pl.BlockSpec((blk, 128), lambda i: (i, 0)),
    )(x, sums, W1, b1r, W2p, b2r)


def kernel(x, emb, W1, b1, W2, b2):
    x = x.astype(jnp.int32)
    nc = W2.shape[1]
    x_lin = jax.lax.optimization_barrier(x.reshape(B * L // 128, 128))
    emb_p = _tc_pack(jnp.swapaxes(emb, 0, 1))
    sums = _get_sc_sums()(x_lin.reshape(-1), emb_p).reshape(B, D)
    W2p = jnp.pad(W2, ((0, 0), (0, 128 - nc)))
    b2r = jnp.pad(b2, ((0, 128 - nc),)).reshape(1, 128)
    b1r = b1.reshape(1, 128)
    out = _tc_head(x, sums, W1, b1r, W2p, b2r)
    return out[:, :nc]
